# Initial kernel scaffold; baseline (speedup 1.0000x reference)
import functools
import math

import jax
import jax.numpy as jnp
from jax import lax
from jax.experimental import pallas as pl
from jax.experimental.pallas import tpu as pltpu
from jax.experimental.pallas import tpu_sc as plsc

NC, NS, L = 2, 16, 16
NW = NC * NS


def _make_sc_aggregate(n, d, e, chunk):
    nch_total = e // chunk
    nch = nch_total // NW
    rows_per_tile = n // NS

    mesh = plsc.VectorSubcoreMesh(core_axis_name="c", subcore_axis_name="s")

    @functools.partial(
        pl.kernel,
        out_type=jax.ShapeDtypeStruct((NC, n, d), jnp.float32),
        mesh=mesh,
        scratch_types=[
            pltpu.VMEM((nch, chunk), jnp.int32),
            pltpu.VMEM((nch, chunk), jnp.int32),
            pltpu.VMEM((nch, chunk), jnp.float32),
            pltpu.VMEM((chunk, d), jnp.float32),
            pltpu.VMEM_SHARED((n, d), jnp.float32),
            pltpu.SemaphoreType.DMA,
        ],
    )
    def sc_kernel(x2_hbm, src_hbm, dst_hbm, val_hbm, zeros_hbm, part_hbm,
                  src_v, dst_v, val_v, rows_v, acc, sem):
        c = lax.axis_index("c")
        s = lax.axis_index("s")
        wid = s * NC + c

        r0 = s * rows_per_tile
        pltpu.sync_copy(zeros_hbm.at[pl.ds(r0, rows_per_tile)],
                        acc.at[pl.ds(r0, rows_per_tile)])
        plsc.subcore_barrier()

        base = wid * nch
        pltpu.sync_copy(src_hbm.at[pl.ds(base, nch)], src_v)
        pltpu.sync_copy(dst_hbm.at[pl.ds(base, nch)], dst_v)
        pltpu.sync_copy(val_hbm.at[pl.ds(base, nch)], val_v)

        def chunk_body(j, carry):
            pltpu.async_copy(x2_hbm.at[src_v.at[j]], rows_v, sem).wait()

            def edge_body(i, carry2):
                v = val_v[j, i]
                for g in range(d // L):
                    sl = pl.ds(g * L, L)
                    rows_v[i, sl] = rows_v[i, sl] * v
                return carry2
            lax.fori_loop(0, chunk, edge_body, 0, unroll=2)

            pltpu.sync_copy(rows_v, acc.at[dst_v.at[j]], add=True)
            return carry
        lax.fori_loop(0, nch, chunk_body, 0)

        plsc.subcore_barrier()
        pltpu.sync_copy(acc.at[pl.ds(r0, rows_per_tile)],
                        part_hbm.at[c, pl.ds(r0, rows_per_tile)])

    return sc_kernel


def _tc_body(p_ref, w_ref, b_ref, g_ref, be_ref, o_ref):
    y = p_ref[0] + p_ref[1]
    z = lax.dot_general(y, w_ref[...], (((1,), (1,)), ((), ())),
                        preferred_element_type=jnp.float32)
    z = z + b_ref[...]
    mu = jnp.mean(z, axis=-1, keepdims=True)
    zc = z - mu
    var = jnp.mean(zc * zc, axis=-1, keepdims=True)
    zn = zc * lax.rsqrt(var + 1e-5) * g_ref[...] + be_ref[...]
    o_ref[...] = zn * 0.5 * (1.0 + lax.erf(zn * (1.0 / math.sqrt(2.0))))


def kernel(x, edge_index, edge_values, W, b, gamma, beta):
    B, n, d_in = x.shape
    d_out = W.shape[0]
    e = edge_values.shape[0]
    x2 = jnp.transpose(x.astype(jnp.float32), (1, 0, 2)).reshape(n, B * d_in)

    chunk = 80
    src = edge_index[1].reshape(e // chunk, chunk)
    dst = edge_index[0].reshape(e // chunk, chunk)
    val = edge_values.reshape(e // chunk, chunk)
    zeros = jnp.zeros((n, B * d_in), jnp.float32)

    sc = _make_sc_aggregate(n, B * d_in, e, chunk)
    partials = sc(x2, src, dst, val, zeros)

    blk = 2000
    out = pl.pallas_call(
        _tc_body,
        grid=(n // blk,),
        in_specs=[
            pl.BlockSpec((NC, blk, d_in), lambda i: (0, i, 0)),
            pl.BlockSpec((d_out, d_in), lambda i: (0, 0)),
            pl.BlockSpec((1, d_out), lambda i: (0, 0)),
            pl.BlockSpec((1, d_out), lambda i: (0, 0)),
            pl.BlockSpec((1, d_out), lambda i: (0, 0)),
        ],
        out_specs=pl.BlockSpec((blk, d_out), lambda i: (i, 0)),
        out_shape=jax.ShapeDtypeStruct((n, d_out), jnp.float32),
    )(partials, W, b.reshape(1, d_out), gamma.reshape(1, d_out),
      beta.reshape(1, d_out))

    return out.reshape(n, B, d_out).transpose(1, 0, 2)


# speedup vs baseline: 3.6947x; 3.6947x over previous
import functools
import math

import jax
import jax.numpy as jnp
from jax import lax
from jax.experimental import pallas as pl
from jax.experimental.pallas import tpu as pltpu
from jax.experimental.pallas import tpu_sc as plsc

NC, NS, L = 2, 16, 16


def _make_sc_aggregate(n_half, r_pad, d, nch, chunk):
    rows_per_tile = r_pad // NS
    trash = r_pad - 8

    mesh = plsc.VectorSubcoreMesh(core_axis_name="c", subcore_axis_name="s")

    @functools.partial(
        pl.kernel,
        out_type=jax.ShapeDtypeStruct((NC, r_pad, d), jnp.float32),
        mesh=mesh,
        scratch_types=[
            pltpu.VMEM((nch, chunk), jnp.int32),
            pltpu.VMEM((nch, chunk), jnp.int32),
            pltpu.VMEM((1, chunk), jnp.int32),
            pltpu.VMEM((nch, chunk), jnp.float32),
            pltpu.VMEM((chunk, d), jnp.float32),
            pltpu.VMEM_SHARED((r_pad, d), jnp.float32),
            pltpu.SemaphoreType.DMA,
        ],
    )
    def sc_kernel(x2_hbm, src_hbm, dst_hbm, val_hbm, zeros_hbm, part_hbm,
                  src_v, dst_v, ldst_v, val_v, rows_v, acc, sem):
        c = lax.axis_index("c")
        s = lax.axis_index("s")
        row_lo = c * n_half

        r0 = pl.multiple_of(s * rows_per_tile, rows_per_tile)
        pltpu.sync_copy(zeros_hbm.at[pl.ds(r0, rows_per_tile)],
                        acc.at[pl.ds(r0, rows_per_tile)])
        plsc.subcore_barrier()

        pltpu.sync_copy(src_hbm.at[s], src_v)
        pltpu.sync_copy(dst_hbm.at[s], dst_v)
        pltpu.sync_copy(val_hbm.at[s], val_v)

        def chunk_body(j, carry):
            pltpu.async_copy(x2_hbm.at[src_v.at[j]], rows_v, sem).wait()

            for q in range(chunk // L):
                sl = pl.ds(q * L, L)
                dq = dst_v[j, sl] - row_lo
                ok = (dq >= 0) & (dq < n_half)
                ldst_v[0, sl] = jnp.where(ok, dq, trash)

            def grp_body(gq, carry2):
                vals16 = val_v[j, pl.ds(gq * L, L)]
                for i in range(L):
                    v = vals16[i]
                    row = gq * L + i
                    for g in range(d // L):
                        sl = pl.ds(g * L, L)
                        rows_v[row, sl] = rows_v[row, sl] * v
                return carry2
            lax.fori_loop(0, chunk // L, grp_body, 0)

            pltpu.sync_copy(rows_v, acc.at[ldst_v.at[0]], add=True)
            return carry
        lax.fori_loop(0, nch, chunk_body, 0)

        plsc.subcore_barrier()
        pltpu.sync_copy(acc.at[pl.ds(r0, rows_per_tile)],
                        part_hbm.at[c, pl.ds(r0, rows_per_tile)])

    return sc_kernel


def _tc_body(p_ref, w_ref, b_ref, g_ref, be_ref, o_ref):
    z = lax.dot_general(p_ref[0], w_ref[...], (((1,), (1,)), ((), ())),
                        preferred_element_type=jnp.float32)
    z = z + b_ref[...]
    mu = jnp.mean(z, axis=-1, keepdims=True)
    zc = z - mu
    var = jnp.mean(zc * zc, axis=-1, keepdims=True)
    zn = zc * lax.rsqrt(var + 1e-5) * g_ref[...] + be_ref[...]
    o_ref[...] = zn * 0.5 * (1.0 + lax.erf(zn * (1.0 / math.sqrt(2.0))))


def kernel(x, edge_index, edge_values, W, b, gamma, beta):
    B, n, d_in = x.shape
    d = B * d_in
    d_out = W.shape[0]
    e = edge_values.shape[0]
    x2 = jnp.transpose(x.astype(jnp.float32), (1, 0, 2)).reshape(n, d)

    chunk = 128
    nch = -(-e // (NS * chunk))
    e_pad = NS * nch * chunk
    src = jnp.concatenate(
        [edge_index[1], jnp.zeros((e_pad - e,), jnp.int32)]).reshape(
            NS, nch, chunk)
    dst = jnp.concatenate(
        [edge_index[0], jnp.zeros((e_pad - e,), jnp.int32)]).reshape(
            NS, nch, chunk)
    val = jnp.concatenate(
        [edge_values, jnp.zeros((e_pad - e,), jnp.float32)]).reshape(
            NS, nch, chunk)

    n_half = n // NC
    r_pad = ((n_half + 16 + NS * 8 - 1) // (NS * 8)) * (NS * 8)
    zeros = jnp.zeros((r_pad, d), jnp.float32)

    sc = _make_sc_aggregate(n_half, r_pad, d, nch, chunk)
    partials = sc(x2, src, dst, val, zeros)

    blk = 1000
    nblk_half = n_half // blk
    out = pl.pallas_call(
        _tc_body,
        grid=(n // blk,),
        in_specs=[
            pl.BlockSpec((1, blk, d),
                         lambda i: (i // nblk_half, i % nblk_half, 0)),
            pl.BlockSpec((d_out, d), lambda i: (0, 0)),
            pl.BlockSpec((1, d_out), lambda i: (0, 0)),
            pl.BlockSpec((1, d_out), lambda i: (0, 0)),
            pl.BlockSpec((1, d_out), lambda i: (0, 0)),
        ],
        out_specs=pl.BlockSpec((blk, d_out), lambda i: (i, 0)),
        out_shape=jax.ShapeDtypeStruct((n, d_out), jnp.float32),
    )(partials, W, b.reshape(1, d_out), gamma.reshape(1, d_out),
      beta.reshape(1, d_out))

    return out.reshape(n, B, d_out).transpose(1, 0, 2)


# speedup vs baseline: 4.7604x; 1.2884x over previous
import functools
import math

import jax
import jax.numpy as jnp
from jax import lax
from jax.experimental import pallas as pl
from jax.experimental.pallas import tpu as pltpu
from jax.experimental.pallas import tpu_sc as plsc

NC, NS, L = 2, 16, 16


def _make_sc_aggregate(n_half, r_pad, d, nch, chunk):
    rows_per_tile = r_pad // NS
    trash = r_pad - 8

    mesh = plsc.VectorSubcoreMesh(core_axis_name="c", subcore_axis_name="s")

    @functools.partial(
        pl.kernel,
        out_type=jax.ShapeDtypeStruct((NC, r_pad, d), jnp.float32),
        mesh=mesh,
        scratch_types=[
            pltpu.VMEM((nch, chunk), jnp.int32),
            pltpu.VMEM((nch, chunk), jnp.int32),
            pltpu.VMEM((2, chunk // 2), jnp.int32),
            pltpu.VMEM((nch, chunk), jnp.float32),
            pltpu.VMEM((chunk // 2, d), jnp.float32),
            pltpu.VMEM((chunk // 2, d), jnp.float32),
            pltpu.VMEM_SHARED((r_pad, d), jnp.float32),
            pltpu.SemaphoreType.DMA,
            pltpu.SemaphoreType.DMA,
        ],
    )
    def sc_kernel(x2_hbm, src_hbm, dst_hbm, val_hbm, zeros_hbm, part_hbm,
                  src_v, dst_v, ldst_v, val_v, rows0_v, rows1_v, acc,
                  sem0, sem1):
        c = lax.axis_index("c")
        s = lax.axis_index("s")
        row_lo = c * n_half

        r0 = pl.multiple_of(s * rows_per_tile, rows_per_tile)
        pltpu.sync_copy(zeros_hbm.at[pl.ds(r0, rows_per_tile)],
                        acc.at[pl.ds(r0, rows_per_tile)])
        plsc.subcore_barrier()

        pltpu.sync_copy(src_hbm.at[s], src_v)
        pltpu.sync_copy(dst_hbm.at[s], dst_v)
        pltpu.sync_copy(val_hbm.at[s], val_v)

        half = chunk // 2
        bufs = (rows0_v, rows1_v)
        sems = (sem0, sem1)

        def start_gather(j, h, buf, sem):
            pltpu.async_copy(x2_hbm.at[src_v.at[j, pl.ds(h * half, half)]],
                             buf, sem)

        def process(j, h, buf):
            for q in range(half // L):
                sl = pl.ds(h * half + q * L, L)
                dq = dst_v[j, sl] - row_lo
                ok = (dq >= 0) & (dq < n_half)
                ldst_v[h, pl.ds(q * L, L)] = jnp.where(ok, dq, trash)

            for gq in range(half // L):
                vals16 = val_v[j, pl.ds(h * half + gq * L, L)]
                for i in range(L):
                    v = vals16[i]
                    row = gq * L + i
                    for g in range(d // L):
                        sl = pl.ds(g * L, L)
                        buf[row, sl] = buf[row, sl] * v

            pltpu.sync_copy(buf, acc.at[ldst_v.at[h]], add=True)

        start_gather(0, 0, bufs[0], sems[0])

        def chunk_body(j, carry):
            start_gather(j, 1, bufs[1], sems[1])
            pltpu.make_async_copy(x2_hbm.at[src_v.at[j, pl.ds(0, half)]],
                                  bufs[0], sems[0]).wait()
            process(j, 0, bufs[0])
            @pl.when(j < nch - 1)
            def _():
                start_gather(j + 1, 0, bufs[0], sems[0])
            pltpu.make_async_copy(x2_hbm.at[src_v.at[j, pl.ds(half, half)]],
                                  bufs[1], sems[1]).wait()
            process(j, 1, bufs[1])
            return carry
        lax.fori_loop(0, nch, chunk_body, 0)

        plsc.subcore_barrier()
        pltpu.sync_copy(acc.at[pl.ds(r0, rows_per_tile)],
                        part_hbm.at[c, pl.ds(r0, rows_per_tile)])

    return sc_kernel


def _tc_body(p_ref, w_ref, b_ref, g_ref, be_ref, o_ref):
    z = lax.dot_general(p_ref[0], w_ref[...], (((1,), (1,)), ((), ())),
                        preferred_element_type=jnp.float32)
    z = z + b_ref[...]
    mu = jnp.mean(z, axis=-1, keepdims=True)
    zc = z - mu
    var = jnp.mean(zc * zc, axis=-1, keepdims=True)
    zn = zc * lax.rsqrt(var + 1e-5) * g_ref[...] + be_ref[...]
    o_ref[...] = zn * 0.5 * (1.0 + lax.erf(zn * (1.0 / math.sqrt(2.0))))


def kernel(x, edge_index, edge_values, W, b, gamma, beta):
    B, n, d_in = x.shape
    d = B * d_in
    d_out = W.shape[0]
    e = edge_values.shape[0]
    x2 = jnp.transpose(x.astype(jnp.float32), (1, 0, 2)).reshape(n, d)

    chunk = 128
    nch = -(-e // (NS * chunk))
    e_pad = NS * nch * chunk
    src = jnp.concatenate(
        [edge_index[1], jnp.zeros((e_pad - e,), jnp.int32)]).reshape(
            NS, nch, chunk)
    dst = jnp.concatenate(
        [edge_index[0], jnp.zeros((e_pad - e,), jnp.int32)]).reshape(
            NS, nch, chunk)
    val = jnp.concatenate(
        [edge_values, jnp.zeros((e_pad - e,), jnp.float32)]).reshape(
            NS, nch, chunk)

    n_half = n // NC
    r_pad = ((n_half + 16 + NS * 8 - 1) // (NS * 8)) * (NS * 8)
    zeros = jnp.zeros((r_pad, d), jnp.float32)

    sc = _make_sc_aggregate(n_half, r_pad, d, nch, chunk)
    partials = sc(x2, src, dst, val, zeros)

    blk = 1000
    nblk_half = n_half // blk
    out = pl.pallas_call(
        _tc_body,
        grid=(n // blk,),
        in_specs=[
            pl.BlockSpec((1, blk, d),
                         lambda i: (i // nblk_half, i % nblk_half, 0)),
            pl.BlockSpec((d_out, d), lambda i: (0, 0)),
            pl.BlockSpec((1, d_out), lambda i: (0, 0)),
            pl.BlockSpec((1, d_out), lambda i: (0, 0)),
            pl.BlockSpec((1, d_out), lambda i: (0, 0)),
        ],
        out_specs=pl.BlockSpec((blk, d_out), lambda i: (i, 0)),
        out_shape=jax.ShapeDtypeStruct((n, d_out), jnp.float32),
    )(partials, W, b.reshape(1, d_out), gamma.reshape(1, d_out),
      beta.reshape(1, d_out))

    return out.reshape(n, B, d_out).transpose(1, 0, 2)
